# baseline (speedup 1.0000x reference)
import functools

import jax
import jax.numpy as jnp
from jax.experimental import pallas as pl
from jax.experimental.pallas import tpu as pltpu

_BLK = 128


def _gcn_kernel(inter_ref, adj1_ref, adj2_ref, e1_ref, e2_ref,
                m_ref, d_ref, p_ref, *, half):
    i = pl.program_id(0)
    y1 = jnp.dot(adj1_ref[...], e1_ref[...], preferred_element_type=jnp.float32)
    y2 = jnp.dot(adj2_ref[...], e2_ref[...], preferred_element_type=jnp.float32)
    t1 = 4.0 * jnp.maximum(y1, 0.0)
    t2 = 4.0 * jnp.maximum(y2, 0.0)

    @pl.when(i < half)
    def _():
        d_ref[...] = t1
        p_ref[...] = t2

    @pl.when(i >= half)
    def _():
        w = inter_ref[0]
        m_ref[...] = w * t1 + (1.0 - w) * t2


def kernel(adj1, adj2, dEmbed, mEmbed, pEmbed, inter):
    e1 = jnp.concatenate([dEmbed, mEmbed], axis=0)
    e2 = jnp.concatenate([pEmbed, mEmbed], axis=0)
    n = adj1.shape[0]
    d = dEmbed.shape[0]
    m = mEmbed.shape[0]
    p = pEmbed.shape[0]
    f = dEmbed.shape[1]
    blk = _BLK
    grid = n // blk
    half = d // blk

    m_out, d_out, p_out = pl.pallas_call(
        functools.partial(_gcn_kernel, half=half),
        grid=(grid,),
        in_specs=[
            pl.BlockSpec(memory_space=pltpu.SMEM),
            pl.BlockSpec((blk, n), lambda i: (i, 0)),
            pl.BlockSpec((blk, n), lambda i: (i, 0)),
            pl.BlockSpec((n, f), lambda i: (0, 0)),
            pl.BlockSpec((n, f), lambda i: (0, 0)),
        ],
        out_specs=[
            pl.BlockSpec((blk, f), lambda i: (jnp.maximum(i - half, 0), 0)),
            pl.BlockSpec((blk, f), lambda i: (jnp.minimum(i, half - 1), 0)),
            pl.BlockSpec((blk, f), lambda i: (jnp.minimum(i, half - 1), 0)),
        ],
        out_shape=[
            jax.ShapeDtypeStruct((m, f), jnp.float32),
            jax.ShapeDtypeStruct((d, f), jnp.float32),
            jax.ShapeDtypeStruct((p, f), jnp.float32),
        ],
    )(inter, adj1, adj2, e1, e2)
    return (m_out, d_out, p_out)


# speedup vs baseline: 1.1906x; 1.1906x over previous
import functools

import jax
import jax.numpy as jnp
from jax.experimental import pallas as pl
from jax.experimental.pallas import tpu as pltpu

_BLK = 256


def _gcn_kernel(inter_ref, adj1_ref, adj2_ref, e1_ref, e2_ref,
                o1_ref, o2_ref, *, half):
    i = pl.program_id(0)
    y1 = jnp.dot(adj1_ref[...], e1_ref[...], preferred_element_type=jnp.float32)
    y2 = jnp.dot(adj2_ref[...], e2_ref[...], preferred_element_type=jnp.float32)
    t1 = 4.0 * jnp.maximum(y1, 0.0)
    t2 = 4.0 * jnp.maximum(y2, 0.0)
    o1_ref[...] = t1

    @pl.when(i < half)
    def _():
        o2_ref[...] = t2

    @pl.when(i >= half)
    def _():
        w = inter_ref[0]
        o2_ref[...] = w * t1 + (1.0 - w) * t2


def kernel(adj1, adj2, dEmbed, mEmbed, pEmbed, inter):
    e1 = jnp.concatenate([dEmbed, mEmbed], axis=0)
    e2 = jnp.concatenate([pEmbed, mEmbed], axis=0)
    n = adj1.shape[0]
    d = dEmbed.shape[0]
    p = pEmbed.shape[0]
    f = dEmbed.shape[1]
    blk = _BLK
    grid = n // blk
    half = d // blk

    o1, o2 = pl.pallas_call(
        functools.partial(_gcn_kernel, half=half),
        grid=(grid,),
        in_specs=[
            pl.BlockSpec(memory_space=pltpu.SMEM),
            pl.BlockSpec((blk, n), lambda i: (i, 0)),
            pl.BlockSpec((blk, n), lambda i: (i, 0)),
            pl.BlockSpec((n, f), lambda i: (0, 0)),
            pl.BlockSpec((n, f), lambda i: (0, 0)),
        ],
        out_specs=[
            pl.BlockSpec((blk, f), lambda i: (i, 0)),
            pl.BlockSpec((blk, f), lambda i: (i, 0)),
        ],
        out_shape=[
            jax.ShapeDtypeStruct((n, f), jnp.float32),
            jax.ShapeDtypeStruct((n, f), jnp.float32),
        ],
    )(inter, adj1, adj2, e1, e2)
    return (o2[p:], o1[:d], o2[:p])


# speedup vs baseline: 1.2034x; 1.0107x over previous
import functools

import jax
import jax.numpy as jnp
from jax.experimental import pallas as pl
from jax.experimental.pallas import tpu as pltpu

_BLK = 256


def _gcn_kernel(inter_ref, adj1_ref, adj2_ref, e1_ref, e2_ref,
                o1_ref, o2_ref, *, half):
    i = pl.program_id(0)
    a1 = adj1_ref[...].astype(jnp.bfloat16)
    a2 = adj2_ref[...].astype(jnp.bfloat16)
    y1 = jnp.dot(a1, e1_ref[...].astype(jnp.bfloat16),
                 preferred_element_type=jnp.float32)
    y2 = jnp.dot(a2, e2_ref[...].astype(jnp.bfloat16),
                 preferred_element_type=jnp.float32)
    t1 = 4.0 * jnp.maximum(y1, 0.0)
    t2 = 4.0 * jnp.maximum(y2, 0.0)
    o1_ref[...] = t1

    @pl.when(i < half)
    def _():
        o2_ref[...] = t2

    @pl.when(i >= half)
    def _():
        w = inter_ref[0]
        o2_ref[...] = w * t1 + (1.0 - w) * t2


def kernel(adj1, adj2, dEmbed, mEmbed, pEmbed, inter):
    e1 = jnp.concatenate([dEmbed, mEmbed], axis=0)
    e2 = jnp.concatenate([pEmbed, mEmbed], axis=0)
    n = adj1.shape[0]
    d = dEmbed.shape[0]
    p = pEmbed.shape[0]
    f = dEmbed.shape[1]
    blk = _BLK
    grid = n // blk
    half = d // blk

    o1, o2 = pl.pallas_call(
        functools.partial(_gcn_kernel, half=half),
        grid=(grid,),
        in_specs=[
            pl.BlockSpec(memory_space=pltpu.SMEM),
            pl.BlockSpec((blk, n), lambda i: (i, 0)),
            pl.BlockSpec((blk, n), lambda i: (i, 0)),
            pl.BlockSpec((n, f), lambda i: (0, 0)),
            pl.BlockSpec((n, f), lambda i: (0, 0)),
        ],
        out_specs=[
            pl.BlockSpec((blk, f), lambda i: (i, 0)),
            pl.BlockSpec((blk, f), lambda i: (i, 0)),
        ],
        out_shape=[
            jax.ShapeDtypeStruct((n, f), jnp.float32),
            jax.ShapeDtypeStruct((n, f), jnp.float32),
        ],
    )(inter, adj1, adj2, e1, e2)
    return (o2[p:], o1[:d], o2[:p])
